# Initial kernel scaffold; baseline (speedup 1.0000x reference)
import jax
import jax.numpy as jnp
from jax.experimental import pallas as pl


def kernel(z, edge_index, W0, b0, W1, b1, W2, b2, r_mlp_W1, r_mlp_b1, r_mlp_W2, r_mlp_b2, t_mlp_W1, t_mlp_b1, t_mlp_W2, t_mlp_b2):
    raise NotImplementedError("write your pallas kernel here")


# speedup vs baseline: 5.8437x; 5.8437x over previous
import functools

import jax
import jax.numpy as jnp
from jax import lax
from jax.experimental import pallas as pl
from jax.experimental.pallas import tpu as pltpu
from jax.experimental.pallas import tpu_sc as plsc

NC = 2
NS = 16
NW = NC * NS
LANES = 16
CH = 128
BR = 1024
BE = 4096

f32 = jnp.float32


def _sc_mesh():
    return plsc.VectorSubcoreMesh(core_axis_name="c", subcore_axis_name="s",
                                  num_cores=NC, num_subcores=NS)


_SC_PARAMS = pltpu.CompilerParams(needs_layout_passes=False)


def _deg_kernel(E_pad, NP):
    EW = E_pad // NW
    nch = EW // CH

    @functools.partial(
        pl.kernel,
        out_type=jax.ShapeDtypeStruct((NW, NP), f32),
        mesh=_sc_mesh(),
        compiler_params=_SC_PARAMS,
        scratch_types=[
            pltpu.VMEM((NP,), f32),
            pltpu.VMEM((CH,), jnp.int32),
        ],
    )
    def kdeg(dstp, out, degl, idxb):
        cid = lax.axis_index("c")
        sid = lax.axis_index("s")
        wid = cid * NS + sid
        zero16 = jnp.zeros((LANES,), f32)
        ones16 = jnp.ones((LANES,), f32)

        def zbody(i, c):
            degl[pl.ds(i * LANES, LANES)] = zero16
            return c
        lax.fori_loop(0, NP // LANES, zbody, 0)

        def cbody(i, c):
            base = wid * EW + i * CH
            pltpu.sync_copy(dstp.at[pl.ds(base, CH)], idxb)
            for j in range(CH // LANES):
                iv = idxb[pl.ds(j * LANES, LANES)]
                plsc.addupdate_scatter(degl, [iv], ones16)
            return c
        lax.fori_loop(0, nch, cbody, 0)

        pltpu.sync_copy(degl, out.at[wid])

    return kdeg


def _conv_scatter_kernel(E_pad, NP, F):
    EW = E_pad // NW
    nch = EW // CH
    rpt = NP // NS

    @functools.partial(
        pl.kernel,
        out_type=jax.ShapeDtypeStruct((NC, NP, F), f32),
        mesh=_sc_mesh(),
        compiler_params=_SC_PARAMS,
        scratch_types=[
            pltpu.VMEM((CH,), jnp.int32),
            pltpu.VMEM((CH,), jnp.int32),
            pltpu.VMEM((CH, F), f32),
            pltpu.VMEM_SHARED((NP, F), f32),
            pltpu.SemaphoreType.DMA,
        ],
    )
    def kconv(hp, srcp, dstp, zinit, out, idx_s, idx_d, rows, acc, sem):
        cid = lax.axis_index("c")
        sid = lax.axis_index("s")
        wid = cid * NS + sid
        pltpu.sync_copy(zinit.at[pl.ds(sid * rpt, rpt)],
                        acc.at[pl.ds(sid * rpt, rpt)])
        plsc.subcore_barrier()

        def cbody(i, c):
            base = wid * EW + i * CH
            pltpu.sync_copy(srcp.at[pl.ds(base, CH)], idx_s)
            pltpu.sync_copy(dstp.at[pl.ds(base, CH)], idx_d)
            pltpu.async_copy(hp.at[idx_s], rows, sem).wait()
            pltpu.sync_copy(rows, acc.at[idx_d], add=True)
            return c
        lax.fori_loop(0, nch, cbody, 0)

        plsc.subcore_barrier()
        pltpu.sync_copy(acc.at[pl.ds(sid * rpt, rpt)],
                        out.at[cid, pl.ds(sid * rpt, rpt)])

    return kconv


def _edge_kernel(E_pad, NP):
    EW = E_pad // NW
    nch = EW // CH
    D = 256

    @functools.partial(
        pl.kernel,
        out_type=(jax.ShapeDtypeStruct((E_pad, 128), f32),
                  jax.ShapeDtypeStruct((E_pad, 128), f32)),
        mesh=_sc_mesh(),
        compiler_params=_SC_PARAMS,
        scratch_types=[
            pltpu.VMEM((CH,), jnp.int32),
            pltpu.VMEM((CH,), jnp.int32),
            pltpu.VMEM((CH, D), f32),
            pltpu.VMEM((CH, D), f32),
            pltpu.SemaphoreType.DMA,
            pltpu.SemaphoreType.DMA,
        ],
    )
    def kedge(us, ud, srcp, dstp, gs_out, gd_out,
              idx_s, idx_d, sv, dv, sem_s, sem_d):
        cid = lax.axis_index("c")
        sid = lax.axis_index("s")
        wid = cid * NS + sid

        def cbody(i, c):
            base = wid * EW + i * CH
            pltpu.sync_copy(srcp.at[pl.ds(base, CH)], idx_s)
            pltpu.sync_copy(dstp.at[pl.ds(base, CH)], idx_d)
            cp_s = pltpu.async_copy(us.at[idx_s], sv, sem_s)
            cp_d = pltpu.async_copy(ud.at[idx_d], dv, sem_d)
            cp_s.wait()
            cp_d.wait()

            def ebody(e, c2):
                acc = jnp.zeros((LANES,), f32)
                for j in range(128 // LANES):
                    sb = sv[e, pl.ds(j * LANES, LANES)]
                    db = dv[e, pl.ds(j * LANES, LANES)]
                    t = sb - db
                    acc = acc + t * t
                sv[e, pl.ds(192, LANES)] = acc
                return c2
            lax.fori_loop(0, CH, ebody, 0)

            pltpu.sync_copy(sv.at[:, pl.ds(128, 128)], gs_out.at[pl.ds(base, CH)])
            pltpu.sync_copy(dv.at[:, pl.ds(128, 128)], gd_out.at[pl.ds(base, CH)])
            return c
        lax.fori_loop(0, nch, cbody, 0)

    return kedge


def _t0_kernel(NP, Z, HID):
    grid = (NP // BR,)

    def body(degp, zp, w0, out_h, out_dis):
        ones = jnp.ones((NW, 1), f32)
        dis = lax.rsqrt(
            lax.dot_general(degp[...], ones, (((0,), (0,)), ((), ())),
                            preferred_element_type=f32) + 1.0)
        out_dis[...] = dis
        out_h[...] = jnp.dot(zp[...], w0[...],
                             preferred_element_type=f32) * dis

    return pl.pallas_call(
        body,
        grid=grid,
        in_specs=[
            pl.BlockSpec((NW, BR), lambda i: (0, i)),
            pl.BlockSpec((BR, Z), lambda i: (i, 0)),
            pl.BlockSpec((Z, HID), lambda i: (0, 0)),
        ],
        out_specs=[
            pl.BlockSpec((BR, HID), lambda i: (i, 0)),
            pl.BlockSpec((BR, 1), lambda i: (i, 0)),
        ],
        out_shape=[
            jax.ShapeDtypeStruct((NP, HID), f32),
            jax.ShapeDtypeStruct((NP, 1), f32),
        ],
    )


def _layer_kernel(NP, Fin, Fout):
    grid = (NP // BR,)

    def body(p, hp, b, w, dis, out):
        pb = p[...]
        x = dis[...] * (pb[0] + pb[1] + hp[...]) + b[...]
        x = jnp.maximum(x, 0.0)
        out[...] = jnp.dot(x, w[...], preferred_element_type=f32) * dis[...]

    return pl.pallas_call(
        body,
        grid=grid,
        in_specs=[
            pl.BlockSpec((NC, BR, Fin), lambda i: (0, i, 0)),
            pl.BlockSpec((BR, Fin), lambda i: (i, 0)),
            pl.BlockSpec((1, Fin), lambda i: (0, 0)),
            pl.BlockSpec((Fin, Fout), lambda i: (0, 0)),
            pl.BlockSpec((BR, 1), lambda i: (i, 0)),
        ],
        out_specs=pl.BlockSpec((BR, Fout), lambda i: (i, 0)),
        out_shape=jax.ShapeDtypeStruct((NP, Fout), f32),
    )


def _pack_kernel(NP, Z, G):
    grid = (NP // BR,)
    HID = 2 * G

    def body(p, hp, b, zp, dis, out_us, out_ud):
        pb = p[...]
        gf = dis[...] * (pb[0] + pb[1] + hp[...]) + b[...]
        g = gf[:, :G]
        zb = zp[...]
        zeros = jnp.zeros((BR, G), f32)
        out_us[...] = jnp.concatenate([zb + 1e-6, g, zeros], axis=1)
        out_ud[...] = jnp.concatenate([zb, zeros, g], axis=1)

    return pl.pallas_call(
        body,
        grid=grid,
        in_specs=[
            pl.BlockSpec((NC, BR, HID), lambda i: (0, i, 0)),
            pl.BlockSpec((BR, HID), lambda i: (i, 0)),
            pl.BlockSpec((1, HID), lambda i: (0, 0)),
            pl.BlockSpec((BR, Z), lambda i: (i, 0)),
            pl.BlockSpec((BR, 1), lambda i: (i, 0)),
        ],
        out_specs=[
            pl.BlockSpec((BR, Z + HID), lambda i: (i, 0)),
            pl.BlockSpec((BR, Z + HID), lambda i: (i, 0)),
        ],
        out_shape=[
            jax.ShapeDtypeStruct((NP, Z + HID), f32),
            jax.ShapeDtypeStruct((NP, Z + HID), f32),
        ],
    )


def _head_kernel(E_pad, G, HID):
    grid = (E_pad // BE,)

    def body(gs, gd, wr1, br1, wr2, br2, wt1, bt1, wt2, bt2, out):
        gsb = gs[...]
        gdb = gd[...]
        inp = jnp.concatenate([gsb[:, :G], gdb[:, G:2 * G]], axis=1)
        sp = gsb[:, G:G + LANES]
        dist = -jnp.sqrt(jnp.sum(sp, axis=1, keepdims=True))

        def head(w1, b1, w2, b2):
            h = jnp.dot(inp, w1[...], preferred_element_type=f32) + b1[...]
            h = jnp.maximum(h, 0.2 * h)
            return jnp.dot(h, w2[...], preferred_element_type=f32) + b2[...]

        r = head(wr1, br1, wr2, br2)
        t = head(wt1, bt1, wt2, bt2)
        out[...] = jax.nn.sigmoid((dist - r) / t)

    return pl.pallas_call(
        body,
        grid=grid,
        in_specs=[
            pl.BlockSpec((BE, 128), lambda i: (i, 0)),
            pl.BlockSpec((BE, 128), lambda i: (i, 0)),
            pl.BlockSpec((2 * G, HID), lambda i: (0, 0)),
            pl.BlockSpec((1, HID), lambda i: (0, 0)),
            pl.BlockSpec((HID, 1), lambda i: (0, 0)),
            pl.BlockSpec((1, 1), lambda i: (0, 0)),
            pl.BlockSpec((2 * G, HID), lambda i: (0, 0)),
            pl.BlockSpec((1, HID), lambda i: (0, 0)),
            pl.BlockSpec((HID, 1), lambda i: (0, 0)),
            pl.BlockSpec((1, 1), lambda i: (0, 0)),
        ],
        out_specs=pl.BlockSpec((BE, 1), lambda i: (i, 0)),
        out_shape=jax.ShapeDtypeStruct((E_pad, 1), f32),
    )


def kernel(z, edge_index, W0, b0, W1, b1, W2, b2,
           r_mlp_W1, r_mlp_b1, r_mlp_W2, r_mlp_b2,
           t_mlp_W1, t_mlp_b1, t_mlp_W2, t_mlp_b2):
    N, Z = z.shape
    E = edge_index.shape[1]
    HID = W0.shape[1]
    G = W2.shape[1]

    NP = -(-(N + 1) // BR) * BR
    EW = -(-E // (NW * CH)) * CH
    E_pad = NW * EW

    zp = jnp.concatenate([z, jnp.zeros((NP - N, Z), f32)], axis=0)
    pad = jnp.full((E_pad - E,), N, jnp.int32)
    srcp = jnp.concatenate([edge_index[0], pad])
    dstp = jnp.concatenate([edge_index[1], pad])
    zinit_h = jnp.zeros((NP, HID), f32)
    W2p = jnp.concatenate([W2, jnp.zeros((HID, HID - G), f32)], axis=1)
    b2p = jnp.concatenate([b2, jnp.zeros((HID - G,), f32)])

    degp = _deg_kernel(E_pad, NP)(dstp)
    hp0, dis = _t0_kernel(NP, Z, HID)(degp, zp, W0)

    p0 = _conv_scatter_kernel(E_pad, NP, HID)(hp0, srcp, dstp, zinit_h)
    hp1 = _layer_kernel(NP, HID, HID)(p0, hp0, b0.reshape(1, HID), W1, dis)

    p1 = _conv_scatter_kernel(E_pad, NP, HID)(hp1, srcp, dstp, zinit_h)
    hp2 = _layer_kernel(NP, HID, HID)(p1, hp1, b1.reshape(1, HID), W2p, dis)

    p2 = _conv_scatter_kernel(E_pad, NP, HID)(hp2, srcp, dstp, zinit_h)
    us, ud = _pack_kernel(NP, Z, G)(p2, hp2, b2p.reshape(1, HID), zp, dis)

    gs, gd = _edge_kernel(E_pad, NP)(us, ud, srcp, dstp)

    probs = _head_kernel(E_pad, G, HID)(
        gs, gd,
        r_mlp_W1, r_mlp_b1.reshape(1, HID), r_mlp_W2, r_mlp_b2.reshape(1, 1),
        t_mlp_W1, t_mlp_b1.reshape(1, HID), t_mlp_W2, t_mlp_b2.reshape(1, 1))
    return probs.reshape(-1)[:E]
